# baseline (speedup 1.0000x reference)
import functools

import jax
import jax.numpy as jnp
from jax import lax
from jax.experimental import pallas as pl
from jax.experimental.pallas import tpu as pltpu
from jax.experimental.pallas import tpu_sc as plsc

U1 = 25000
N = 50000
NP = 50048
D = 64
HD = 32
E = 800000
B = 16384
NC = 2
NT = 16
C = 128
J = 14
K = 28
JB = 8
EP = K * J * C
RPT = NP // NT
ZR = 136
BP = B // NT

_ROWS = 1000


def _mlp_body(x_ref, ws_ref, bs_ref, wr1_ref, wr2_ref, br_ref, out_ref):
    x = x_ref[...]
    share = jnp.maximum(
        jnp.dot(x, ws_ref[...], preferred_element_type=jnp.float32) + bs_ref[...],
        0.0)
    y = (jnp.dot(x, wr1_ref[...], preferred_element_type=jnp.float32)
         + jnp.dot(share, wr2_ref[...], preferred_element_type=jnp.float32)
         + br_ref[...])
    out_ref[...] = jnp.maximum(y, 0.0)


def _mlp(u2e_r, W_s, b_s, W_r, b_r):
    return pl.pallas_call(
        _mlp_body,
        grid=(U1 // _ROWS,),
        in_specs=[
            pl.BlockSpec((_ROWS, D), lambda i: (i, 0)),
            pl.BlockSpec((D, D), lambda i: (0, 0)),
            pl.BlockSpec((D,), lambda i: (0,)),
            pl.BlockSpec((D, D), lambda i: (0, 0)),
            pl.BlockSpec((D, D), lambda i: (0, 0)),
            pl.BlockSpec((D,), lambda i: (0,)),
        ],
        out_specs=pl.BlockSpec((_ROWS, D), lambda i: (i, 0)),
        out_shape=jax.ShapeDtypeStruct((U1, D), jnp.float32),
    )(u2e_r, W_s, b_s, W_r[:D], W_r[D:], b_r)


def _sc_body(t0, src_e, dst_e, vals_e, uidx, iidx, z_hbm,
             usum, isum, t1, t2, t3,
             acc, sidx, didx, vbuf, rows, rows2, rows3,
             gsem, ssem):
    ubuf = rows2
    c = lax.axis_index("c")
    s = lax.axis_index("s")
    base_row = s * RPT

    def scale_rows(j, buf, vb):
        def sr(g, carry):
            vals16 = vb[j, pl.ds(g * 16, 16)]
            for u in range(16):
                r = g * 16 + u
                v = vals16[u]
                buf[r, pl.ds(0, 16)] = buf[r, pl.ds(0, 16)] * v
                buf[r, pl.ds(16, 16)] = buf[r, pl.ds(16, 16)] * v
            return carry
        lax.fori_loop(0, C // 16, sr, 0)

    def layer(t_in, t_out):
        pltpu.sync_copy(z_hbm.at[pl.ds(base_row, RPT)],
                        acc.at[pl.ds(base_row, RPT)])
        plsc.subcore_barrier()

        bufs = (rows, rows2, rows3)

        def ek(k, carry):
            pltpu.sync_copy(src_e.at[c, s, k], sidx)
            pltpu.sync_copy(dst_e.at[s, k], didx)
            pltpu.sync_copy(vals_e.at[s, k], vbuf)
            g = {}
            sc = {}
            g[0] = pltpu.async_copy(t_in.at[sidx.at[0]], bufs[0], gsem)
            g[1] = pltpu.async_copy(t_in.at[sidx.at[1]], bufs[1], gsem)
            for j in range(J):
                g[j].wait()
                if j >= 1:
                    sc[j - 1].wait()
                if j + 2 < J:
                    g[j + 2] = pltpu.async_copy(
                        t_in.at[sidx.at[j + 2]], bufs[(j + 2) % 3], gsem)
                scale_rows(j, bufs[j % 3], vbuf)
                sc[j] = pltpu.async_copy(
                    bufs[j % 3], acc.at[didx.at[j]], ssem, add=True)
            sc[J - 1].wait()
            return carry
        lax.fori_loop(0, K, ek, 0)
        plsc.subcore_barrier()

        pltpu.sync_copy(acc.at[pl.ds(base_row, RPT)],
                        t_out.at[pl.ds(c * NP + base_row, RPT)])
        plsc.subcore_barrier()

    layer(t0, t1)
    layer(t1, t2)
    layer(t2, t3)

    pltpu.sync_copy(uidx.at[c, s], sidx.at[pl.ds(0, JB)])
    pltpu.sync_copy(iidx.at[c, s], didx.at[pl.ds(0, JB)])
    for j in range(JB):
        for ids, shbm in ((sidx, usum), (didx, isum)):
            pltpu.async_copy(t0.at[ids.at[j]], ubuf, gsem).wait()
            for t in (t1, t2, t3):
                pltpu.async_copy(t.at[ids.at[j]], rows, gsem).wait()

                def su(rr, carry):
                    for h in (0, 16):
                        ubuf[rr, pl.ds(h, 16)] = (ubuf[rr, pl.ds(h, 16)]
                                                  + rows[rr, pl.ds(h, 16)])
                    return carry
                lax.fori_loop(0, C, su, 0)
            pltpu.sync_copy(ubuf, shbm.at[c, pl.ds(s * BP + j * C, C)])


_sc_call = pl.kernel(
    _sc_body,
    out_type=(
        jax.ShapeDtypeStruct((NC, B, HD), jnp.float32),
        jax.ShapeDtypeStruct((NC, B, HD), jnp.float32),
        jax.ShapeDtypeStruct((NC * NP, HD), jnp.float32),
        jax.ShapeDtypeStruct((NC * NP, HD), jnp.float32),
        jax.ShapeDtypeStruct((NC * NP, HD), jnp.float32),
    ),
    mesh=plsc.VectorSubcoreMesh(
        core_axis_name="c", subcore_axis_name="s",
        num_cores=NC, num_subcores=NT),
    compiler_params=pltpu.CompilerParams(use_tc_tiling_on_sc=False),
    scratch_types=[
        pltpu.VMEM_SHARED((NP, HD), jnp.float32),
        pltpu.VMEM((J, C), jnp.int32),
        pltpu.VMEM((J, C), jnp.int32),
        pltpu.VMEM((J, C), jnp.float32),
        pltpu.VMEM((C, HD), jnp.float32),
        pltpu.VMEM((C, HD), jnp.float32),
        pltpu.VMEM((C, HD), jnp.float32),
        pltpu.SemaphoreType.DMA,
        pltpu.SemaphoreType.DMA,
    ],
)


_DB = 2048


def _dot_body(u_ref, i_ref, out_ref):
    prod = u_ref[...] * i_ref[...]
    out_ref[...] = jnp.sum(prod, axis=(0, 2)) * (1.0 / 16.0)


def _dot(usum, isum):
    return pl.pallas_call(
        _dot_body,
        grid=(B // _DB,),
        in_specs=[
            pl.BlockSpec((NC, _DB, HD), lambda i: (0, i, 0)),
            pl.BlockSpec((NC, _DB, HD), lambda i: (0, i, 0)),
        ],
        out_specs=pl.BlockSpec((_DB,), lambda i: (i,)),
        out_shape=jax.ShapeDtypeStruct((B,), jnp.float32),
    )(usum, isum)


def kernel(u2e_r, u2e_t, emb_item, W_s, b_s, W_r, b_r, W_t, b_t,
           edge_vals, users, items, edge_index):
    u2e_r_new = _mlp(u2e_r, W_s, b_s, W_r, b_r)
    all_emb = jnp.concatenate([u2e_r_new, emb_item], axis=0)
    zrows = jnp.zeros((NP - N, HD), jnp.float32)
    t0 = jnp.concatenate(
        [all_emb[:, :HD], zrows, all_emb[:, HD:], zrows], axis=0)

    dst = edge_index[0].astype(jnp.int32)
    src = edge_index[1].astype(jnp.int32)
    vals = edge_vals
    pad = NT * EP - E
    offs = jnp.array([0, NP], jnp.int32)
    src_e = (jnp.pad(src, (0, pad)).reshape(NT, K, J, C)[None]
             + offs[:, None, None, None, None])
    dst_e = jnp.pad(dst, (0, pad)).reshape(NT, K, J, C)
    vals_e = jnp.pad(vals, (0, pad)).reshape(NT, K, J, C)

    uidx = (users.astype(jnp.int32)[None, :]
            + offs[:, None]).reshape(NC, NT, JB, C)
    iidx = (items.astype(jnp.int32)[None, :] + U1
            + offs[:, None]).reshape(NC, NT, JB, C)

    z_hbm = jnp.zeros((NP, HD), jnp.float32)
    usum, isum, _t1, _t2, _t3 = _sc_call(
        t0, src_e, dst_e, vals_e, uidx, iidx, z_hbm)
    return _dot(usum, isum)


# speedup vs baseline: 1.0114x; 1.0114x over previous
import functools

import jax
import jax.numpy as jnp
from jax import lax
from jax.experimental import pallas as pl
from jax.experimental.pallas import tpu as pltpu
from jax.experimental.pallas import tpu_sc as plsc

U1 = 25000
N = 50000
NP = 50048
D = 64
HD = 32
E = 800000
B = 16384
NC = 2
NT = 16
C = 128
J = 14
K = 28
JB = 8
EP = K * J * C
RPT = NP // NT
ZR = 136
BP = B // NT

_ROWS = 1000


def _mlp_body(x_ref, ws_ref, bs_ref, wr1_ref, wr2_ref, br_ref, out_ref):
    x = x_ref[...]
    share = jnp.maximum(
        jnp.dot(x, ws_ref[...], preferred_element_type=jnp.float32) + bs_ref[...],
        0.0)
    y = (jnp.dot(x, wr1_ref[...], preferred_element_type=jnp.float32)
         + jnp.dot(share, wr2_ref[...], preferred_element_type=jnp.float32)
         + br_ref[...])
    out_ref[...] = jnp.maximum(y, 0.0)


def _mlp(u2e_r, W_s, b_s, W_r, b_r):
    return pl.pallas_call(
        _mlp_body,
        grid=(U1 // _ROWS,),
        in_specs=[
            pl.BlockSpec((_ROWS, D), lambda i: (i, 0)),
            pl.BlockSpec((D, D), lambda i: (0, 0)),
            pl.BlockSpec((D,), lambda i: (0,)),
            pl.BlockSpec((D, D), lambda i: (0, 0)),
            pl.BlockSpec((D, D), lambda i: (0, 0)),
            pl.BlockSpec((D,), lambda i: (0,)),
        ],
        out_specs=pl.BlockSpec((_ROWS, D), lambda i: (i, 0)),
        out_shape=jax.ShapeDtypeStruct((U1, D), jnp.float32),
    )(u2e_r, W_s, b_s, W_r[:D], W_r[D:], b_r)


def _sc_body(t0, src_e, dst_e, vals_e, uidx, iidx,
             usum, isum, t1, t2, t3,
             acc, sidx, didx, vbuf, rows, rows2, rows3, ubuf, zbuf,
             gsem, ssem):
    c = lax.axis_index("c")
    s = lax.axis_index("s")
    base_row = s * RPT

    def zb(r, carry):
        z = jnp.zeros((16,), jnp.float32)
        zbuf[r, pl.ds(0, 16)] = z
        zbuf[r, pl.ds(16, 16)] = z
        return carry
    lax.fori_loop(0, ZR, zb, 0)

    def scale_rows(j, buf, vb):
        def sr(g, carry):
            vals16 = vb[j, pl.ds(g * 16, 16)]
            for u in range(16):
                r = g * 16 + u
                v = vals16[u]
                buf[r, pl.ds(0, 16)] = buf[r, pl.ds(0, 16)] * v
                buf[r, pl.ds(16, 16)] = buf[r, pl.ds(16, 16)] * v
            return carry
        lax.fori_loop(0, C // 16, sr, 0)

    def layer(t_in, t_out):
        def zc(q, carry):
            pltpu.sync_copy(zbuf, acc.at[pl.ds(base_row + q * ZR, ZR)])
            return carry
        lax.fori_loop(0, RPT // ZR, zc, 0)
        plsc.subcore_barrier()

        bufs = (rows, rows2, rows3)

        def ek(k, carry):
            pltpu.sync_copy(src_e.at[c, s, k], sidx)
            pltpu.sync_copy(dst_e.at[s, k], didx)
            pltpu.sync_copy(vals_e.at[s, k], vbuf)
            g = {}
            sc = {}
            g[0] = pltpu.async_copy(t_in.at[sidx.at[0]], bufs[0], gsem)
            g[1] = pltpu.async_copy(t_in.at[sidx.at[1]], bufs[1], gsem)
            for j in range(J):
                g[j].wait()
                if j >= 1:
                    sc[j - 1].wait()
                if j + 2 < J:
                    g[j + 2] = pltpu.async_copy(
                        t_in.at[sidx.at[j + 2]], bufs[(j + 2) % 3], gsem)
                scale_rows(j, bufs[j % 3], vbuf)
                sc[j] = pltpu.async_copy(
                    bufs[j % 3], acc.at[didx.at[j]], ssem, add=True)
            sc[J - 1].wait()
            return carry
        lax.fori_loop(0, K, ek, 0)
        plsc.subcore_barrier()

        pltpu.sync_copy(acc.at[pl.ds(base_row, RPT)],
                        t_out.at[pl.ds(c * NP + base_row, RPT)])
        plsc.subcore_barrier()

    layer(t0, t1)
    layer(t1, t2)
    layer(t2, t3)

    pltpu.sync_copy(uidx.at[c, s], sidx.at[pl.ds(0, JB)])
    pltpu.sync_copy(iidx.at[c, s], didx.at[pl.ds(0, JB)])
    for j in range(JB):
        for ids, shbm in ((sidx, usum), (didx, isum)):
            pltpu.async_copy(t0.at[ids.at[j]], ubuf, gsem).wait()
            for t in (t1, t2, t3):
                pltpu.async_copy(t.at[ids.at[j]], rows, gsem).wait()

                def su(rr, carry):
                    for h in (0, 16):
                        ubuf[rr, pl.ds(h, 16)] = (ubuf[rr, pl.ds(h, 16)]
                                                  + rows[rr, pl.ds(h, 16)])
                    return carry
                lax.fori_loop(0, C, su, 0)
            pltpu.sync_copy(ubuf, shbm.at[c, pl.ds(s * BP + j * C, C)])


_sc_call = pl.kernel(
    _sc_body,
    out_type=(
        jax.ShapeDtypeStruct((NC, B, HD), jnp.float32),
        jax.ShapeDtypeStruct((NC, B, HD), jnp.float32),
        jax.ShapeDtypeStruct((NC * NP, HD), jnp.float32),
        jax.ShapeDtypeStruct((NC * NP, HD), jnp.float32),
        jax.ShapeDtypeStruct((NC * NP, HD), jnp.float32),
    ),
    mesh=plsc.VectorSubcoreMesh(
        core_axis_name="c", subcore_axis_name="s",
        num_cores=NC, num_subcores=NT),
    compiler_params=pltpu.CompilerParams(use_tc_tiling_on_sc=False),
    scratch_types=[
        pltpu.VMEM_SHARED((NP, HD), jnp.float32),
        pltpu.VMEM((J, C), jnp.int32),
        pltpu.VMEM((J, C), jnp.int32),
        pltpu.VMEM((J, C), jnp.float32),
        pltpu.VMEM((C, HD), jnp.float32),
        pltpu.VMEM((C, HD), jnp.float32),
        pltpu.VMEM((C, HD), jnp.float32),
        pltpu.VMEM((C, HD), jnp.float32),
        pltpu.VMEM((ZR, HD), jnp.float32),
        pltpu.SemaphoreType.DMA,
        pltpu.SemaphoreType.DMA,
    ],
)


_DB = 2048


def _dot_body(u_ref, i_ref, out_ref):
    prod = u_ref[...] * i_ref[...]
    out_ref[...] = jnp.sum(prod, axis=(0, 2)) * (1.0 / 16.0)


def _dot(usum, isum):
    return pl.pallas_call(
        _dot_body,
        grid=(B // _DB,),
        in_specs=[
            pl.BlockSpec((NC, _DB, HD), lambda i: (0, i, 0)),
            pl.BlockSpec((NC, _DB, HD), lambda i: (0, i, 0)),
        ],
        out_specs=pl.BlockSpec((_DB,), lambda i: (i,)),
        out_shape=jax.ShapeDtypeStruct((B,), jnp.float32),
    )(usum, isum)


def kernel(u2e_r, u2e_t, emb_item, W_s, b_s, W_r, b_r, W_t, b_t,
           edge_vals, users, items, edge_index):
    u2e_r_new = _mlp(u2e_r, W_s, b_s, W_r, b_r)
    all_emb = jnp.concatenate([u2e_r_new, emb_item], axis=0)
    zrows = jnp.zeros((NP - N, HD), jnp.float32)
    t0 = jnp.concatenate(
        [all_emb[:, :HD], zrows, all_emb[:, HD:], zrows], axis=0)

    dst = edge_index[0].astype(jnp.int32)
    src = edge_index[1].astype(jnp.int32)
    vals = edge_vals
    pad = NT * EP - E
    offs = jnp.array([0, NP], jnp.int32)
    src_e = (jnp.pad(src, (0, pad)).reshape(NT, K, J, C)[None]
             + offs[:, None, None, None, None])
    dst_e = jnp.pad(dst, (0, pad)).reshape(NT, K, J, C)
    vals_e = jnp.pad(vals, (0, pad)).reshape(NT, K, J, C)

    uidx = (users.astype(jnp.int32)[None, :]
            + offs[:, None]).reshape(NC, NT, JB, C)
    iidx = (items.astype(jnp.int32)[None, :] + U1
            + offs[:, None]).reshape(NC, NT, JB, C)

    usum, isum, _t1, _t2, _t3 = _sc_call(t0, src_e, dst_e, vals_e, uidx, iidx)
    return _dot(usum, isum)


# speedup vs baseline: 1.0488x; 1.0370x over previous
import functools

import jax
import jax.numpy as jnp
from jax import lax
from jax.experimental import pallas as pl
from jax.experimental.pallas import tpu as pltpu
from jax.experimental.pallas import tpu_sc as plsc

U1 = 25000
N = 50000
NP = 50048
D = 64
HD = 32
E = 800000
B = 16384
NC = 2
NT = 16
C = 128
J = 14
K = 28
JB = 8
EP = K * J * C
RPT = NP // NT
ZR = 136
BP = B // NT

_ROWS = 1000


def _mlp_body(x_ref, ws_ref, bs_ref, wr1_ref, wr2_ref, br_ref, out_ref):
    x = x_ref[...]
    share = jnp.maximum(
        jnp.dot(x, ws_ref[...], preferred_element_type=jnp.float32) + bs_ref[...],
        0.0)
    y = (jnp.dot(x, wr1_ref[...], preferred_element_type=jnp.float32)
         + jnp.dot(share, wr2_ref[...], preferred_element_type=jnp.float32)
         + br_ref[...])
    out_ref[...] = jnp.maximum(y, 0.0)


def _mlp(u2e_r, W_s, b_s, W_r, b_r):
    return pl.pallas_call(
        _mlp_body,
        grid=(U1 // _ROWS,),
        in_specs=[
            pl.BlockSpec((_ROWS, D), lambda i: (i, 0)),
            pl.BlockSpec((D, D), lambda i: (0, 0)),
            pl.BlockSpec((D,), lambda i: (0,)),
            pl.BlockSpec((D, D), lambda i: (0, 0)),
            pl.BlockSpec((D, D), lambda i: (0, 0)),
            pl.BlockSpec((D,), lambda i: (0,)),
        ],
        out_specs=pl.BlockSpec((_ROWS, D), lambda i: (i, 0)),
        out_shape=jax.ShapeDtypeStruct((U1, D), jnp.float32),
    )(u2e_r, W_s, b_s, W_r[:D], W_r[D:], b_r)


def _sc_body(t0, src_e, dst_e, vals_e, uidx, iidx,
             usum, isum, t1, t2, t3,
             acc, sidx, didx, vbuf, rows, rows2, rows3, ubuf, zbuf,
             gsem, ssem):
    c = lax.axis_index("c")
    s = lax.axis_index("s")
    base_row = s * RPT

    def zb(r, carry):
        z = jnp.zeros((16,), jnp.float32)
        zbuf[r, pl.ds(0, 16)] = z
        zbuf[r, pl.ds(16, 16)] = z
        return carry
    lax.fori_loop(0, ZR, zb, 0)

    def scale_rows(j, buf, vb):
        def sr(g, carry):
            vals16 = vb[j, pl.ds(g * 16, 16)]
            for u in range(16):
                r = g * 16 + u
                v = vals16[u]
                buf[r, pl.ds(0, 16)] = buf[r, pl.ds(0, 16)] * v
                buf[r, pl.ds(16, 16)] = buf[r, pl.ds(16, 16)] * v
            return carry
        lax.fori_loop(0, C // 16, sr, 0)

    def layer(t_in, t_out):
        def zc(q, carry):
            pltpu.sync_copy(zbuf, acc.at[pl.ds(base_row + q * ZR, ZR)])
            return carry
        lax.fori_loop(0, RPT // ZR, zc, 0)
        plsc.subcore_barrier()

        bufs = (rows, rows2, rows3)

        def ek(k, carry):
            pltpu.sync_copy(src_e.at[c, s, k], sidx)
            pltpu.sync_copy(dst_e.at[s, k], didx)
            pltpu.sync_copy(vals_e.at[s, k], vbuf)
            g = {}
            sc = {}
            g[0] = pltpu.async_copy(t_in.at[sidx.at[0]], bufs[0], gsem)
            g[1] = pltpu.async_copy(t_in.at[sidx.at[1]], bufs[1], gsem)
            for j in range(J):
                g[j].wait()
                if j >= 1:
                    sc[j - 1].wait()
                if j + 2 < J:
                    g[j + 2] = pltpu.async_copy(
                        t_in.at[sidx.at[j + 2]], bufs[(j + 2) % 3], gsem)
                scale_rows(j, bufs[j % 3], vbuf)
                sc[j] = pltpu.async_copy(
                    bufs[j % 3], acc.at[didx.at[j]], ssem, add=True)
            sc[J - 1].wait()
            return carry
        lax.fori_loop(0, K, ek, 0)
        plsc.subcore_barrier()

        pltpu.sync_copy(acc.at[pl.ds(base_row, RPT)],
                        t_out.at[pl.ds(c * NP + base_row, RPT)])
        plsc.subcore_barrier()

    layer(t0, t1)
    layer(t1, t2)
    layer(t2, t3)

    pltpu.sync_copy(uidx.at[c, s], sidx.at[pl.ds(0, JB)])
    pltpu.sync_copy(iidx.at[c, s], didx.at[pl.ds(0, JB)])
    for j in range(JB):
        for ids, shbm in ((sidx, usum), (didx, isum)):
            d0 = pltpu.async_copy(t0.at[ids.at[j]], ubuf, gsem)
            d1 = pltpu.async_copy(t1.at[ids.at[j]], rows, gsem)
            d2 = pltpu.async_copy(t2.at[ids.at[j]], rows2, gsem)
            d3 = pltpu.async_copy(t3.at[ids.at[j]], rows3, gsem)
            d0.wait(); d1.wait(); d2.wait(); d3.wait()

            def su(rr, carry):
                for h in (0, 16):
                    ubuf[rr, pl.ds(h, 16)] = (
                        ubuf[rr, pl.ds(h, 16)] + rows[rr, pl.ds(h, 16)]
                        + rows2[rr, pl.ds(h, 16)] + rows3[rr, pl.ds(h, 16)])
                return carry
            lax.fori_loop(0, C, su, 0)
            pltpu.sync_copy(ubuf, shbm.at[c, pl.ds(s * BP + j * C, C)])


_sc_call = pl.kernel(
    _sc_body,
    out_type=(
        jax.ShapeDtypeStruct((NC, B, HD), jnp.float32),
        jax.ShapeDtypeStruct((NC, B, HD), jnp.float32),
        jax.ShapeDtypeStruct((NC * NP, HD), jnp.float32),
        jax.ShapeDtypeStruct((NC * NP, HD), jnp.float32),
        jax.ShapeDtypeStruct((NC * NP, HD), jnp.float32),
    ),
    mesh=plsc.VectorSubcoreMesh(
        core_axis_name="c", subcore_axis_name="s",
        num_cores=NC, num_subcores=NT),
    compiler_params=pltpu.CompilerParams(use_tc_tiling_on_sc=False),
    scratch_types=[
        pltpu.VMEM_SHARED((NP, HD), jnp.float32),
        pltpu.VMEM((J, C), jnp.int32),
        pltpu.VMEM((J, C), jnp.int32),
        pltpu.VMEM((J, C), jnp.float32),
        pltpu.VMEM((C, HD), jnp.float32),
        pltpu.VMEM((C, HD), jnp.float32),
        pltpu.VMEM((C, HD), jnp.float32),
        pltpu.VMEM((C, HD), jnp.float32),
        pltpu.VMEM((ZR, HD), jnp.float32),
        pltpu.SemaphoreType.DMA,
        pltpu.SemaphoreType.DMA,
    ],
)


_DB = 2048


def _dot_body(u_ref, i_ref, out_ref):
    prod = u_ref[...] * i_ref[...]
    out_ref[...] = jnp.sum(prod, axis=(0, 2)) * (1.0 / 16.0)


def _dot(usum, isum):
    return pl.pallas_call(
        _dot_body,
        grid=(B // _DB,),
        in_specs=[
            pl.BlockSpec((NC, _DB, HD), lambda i: (0, i, 0)),
            pl.BlockSpec((NC, _DB, HD), lambda i: (0, i, 0)),
        ],
        out_specs=pl.BlockSpec((_DB,), lambda i: (i,)),
        out_shape=jax.ShapeDtypeStruct((B,), jnp.float32),
    )(usum, isum)


def kernel(u2e_r, u2e_t, emb_item, W_s, b_s, W_r, b_r, W_t, b_t,
           edge_vals, users, items, edge_index):
    u2e_r_new = _mlp(u2e_r, W_s, b_s, W_r, b_r)
    all_emb = jnp.concatenate([u2e_r_new, emb_item], axis=0)
    zrows = jnp.zeros((NP - N, HD), jnp.float32)
    t0 = jnp.concatenate(
        [all_emb[:, :HD], zrows, all_emb[:, HD:], zrows], axis=0)

    dst = edge_index[0].astype(jnp.int32)
    src = edge_index[1].astype(jnp.int32)
    vals = edge_vals
    pad = NT * EP - E
    offs = jnp.array([0, NP], jnp.int32)
    src_e = (jnp.pad(src, (0, pad)).reshape(NT, K, J, C)[None]
             + offs[:, None, None, None, None])
    dst_e = jnp.pad(dst, (0, pad)).reshape(NT, K, J, C)
    vals_e = jnp.pad(vals, (0, pad)).reshape(NT, K, J, C)

    uidx = (users.astype(jnp.int32)[None, :]
            + offs[:, None]).reshape(NC, NT, JB, C)
    iidx = (items.astype(jnp.int32)[None, :] + U1
            + offs[:, None]).reshape(NC, NT, JB, C)

    usum, isum, _t1, _t2, _t3 = _sc_call(t0, src_e, dst_e, vals_e, uidx, iidx)
    return _dot(usum, isum)
